# Initial kernel scaffold; baseline (speedup 1.0000x reference)
import jax
import jax.numpy as jnp
from jax.experimental import pallas as pl


def kernel(x, edge_index, W1, b1, Wr1, br1, g1, be1, W2, b2, Wr2, br2, g2, be2, w_att, b_att):
    raise NotImplementedError("write your pallas kernel here")


# speedup vs baseline: 6.7867x; 6.7867x over previous
import functools

import jax
import jax.numpy as jnp
from jax import lax
from jax.experimental import pallas as pl
from jax.experimental.pallas import tpu as pltpu
from jax.experimental.pallas import tpu_sc as plsc

N = 10000
E = 320000
D = 128

NC = 2
NS = 16
NW = NC * NS
EPW = E // NW
CH = 80
NCHUNK = EPW // CH
WSTRIDE = 624
WROWS = 640

_SC_MESH = plsc.VectorSubcoreMesh(core_axis_name="c", subcore_axis_name="s")


@functools.partial(
    pl.kernel,
    out_type=jax.ShapeDtypeStruct((NC, N, D), jnp.float32),
    mesh=_SC_MESH,
    scratch_types=[
        pltpu.VMEM((NCHUNK, CH), jnp.int32),
        pltpu.VMEM((NCHUNK, CH), jnp.int32),
        pltpu.VMEM((CH, D), jnp.float32),
        pltpu.VMEM_SHARED((N, D), jnp.float32),
        pltpu.SemaphoreType.DMA,
    ],
)
def _segsum(h_hbm, src_hbm, dst_hbm, out_hbm, src_all, dst_all, rows_v,
            acc_sh, sem):
    c = lax.axis_index("c")
    s = lax.axis_index("s")
    wid = s * NC + c
    row0 = jnp.minimum(s * WSTRIDE, N - WROWS)

    pltpu.sync_copy(src_hbm.at[wid], src_all)
    pltpu.sync_copy(dst_hbm.at[wid], dst_all)

    zero16 = jnp.zeros((16,), jnp.float32)

    def zrow(i, carry):
        for j in range(D // 16):
            rows_v[i, pl.ds(j * 16, 16)] = zero16
        return carry

    lax.fori_loop(0, CH, zrow, 0)
    for k in range(WROWS // CH):
        pltpu.sync_copy(rows_v, acc_sh.at[pl.ds(row0 + k * CH, CH)])
    plsc.subcore_barrier()

    def chunk(j, carry):
        pltpu.async_copy(h_hbm.at[src_all.at[j]], rows_v, sem).wait()
        pltpu.sync_copy(rows_v, acc_sh.at[dst_all.at[j]], add=True)
        return carry

    lax.fori_loop(0, NCHUNK, chunk, 0)
    plsc.subcore_barrier()

    for k in range(WROWS // CH):
        pltpu.sync_copy(acc_sh.at[pl.ds(row0 + k * CH, CH)], rows_v)
        pltpu.sync_copy(rows_v, out_hbm.at[c, pl.ds(row0 + k * CH, CH)])


def _bn_relu_combine(p_ref, h_ref, W_ref, b_ref, Wr_ref, br_ref, g_ref, be_ref):
    agg = p_ref[0] + p_ref[1]
    out = jnp.maximum(
        jnp.dot(agg, W_ref[...], preferred_element_type=jnp.float32)
        + b_ref[...], 0.0)
    res = jnp.maximum(
        jnp.dot(h_ref[...], Wr_ref[...], preferred_element_type=jnp.float32)
        + br_ref[...], 0.0)
    out = out + res
    mu = jnp.mean(out, axis=0, keepdims=True)
    var = jnp.mean((out - mu) ** 2, axis=0, keepdims=True)
    return g_ref[...] * (out - mu) * lax.rsqrt(var + 1e-5) + be_ref[...]


def _dense1_body(p_ref, h_ref, W_ref, b_ref, Wr_ref, br_ref, g_ref, be_ref,
                 o_ref):
    o_ref[...] = _bn_relu_combine(p_ref, h_ref, W_ref, b_ref, Wr_ref, br_ref,
                                  g_ref, be_ref)


def _dense2_body(p_ref, h_ref, W_ref, b_ref, Wr_ref, br_ref, g_ref, be_ref,
                 watt_ref, batt_ref, o_ref):
    h2 = _bn_relu_combine(p_ref, h_ref, W_ref, b_ref, Wr_ref, br_ref, g_ref,
                          be_ref)
    logit = jnp.sum(h2 * watt_ref[...], axis=1, keepdims=True) + batt_ref[...]
    wgt = 1.0 / (1.0 + jnp.exp(-logit))
    hsum = jnp.sum(wgt * h2, axis=0, keepdims=True)
    hmax = jnp.max(h2, axis=0, keepdims=True)
    o_ref[...] = jnp.concatenate([hsum, hmax], axis=1)


_dense1 = pl.pallas_call(
    _dense1_body,
    out_shape=jax.ShapeDtypeStruct((N, D), jnp.float32),
)

_dense2 = pl.pallas_call(
    _dense2_body,
    out_shape=jax.ShapeDtypeStruct((1, 2 * D), jnp.float32),
)


def kernel(x, edge_index, W1, b1, Wr1, br1, g1, be1, W2, b2, Wr2, br2, g2,
           be2, w_att, b_att):
    src = edge_index[0].reshape(NW, NCHUNK, CH)
    dst = edge_index[1].reshape(NW, NCHUNK, CH)
    row = lambda v: v.reshape(1, -1)
    P1 = _segsum(x, src, dst)
    h1 = _dense1(P1, x, W1, row(b1), Wr1, row(br1), row(g1), row(be1))
    P2 = _segsum(h1, src, dst)
    return _dense2(P2, h1, W2, row(b2), Wr2, row(br2), row(g2), row(be2),
                   row(w_att), b_att.reshape(1, 1))
